# baseline (speedup 1.0000x reference)
import jax
import jax.numpy as jnp
from jax import lax
from jax.experimental import pallas as pl
from jax.experimental.pallas import tpu as pltpu
from jax.experimental.pallas import tpu_sc as plsc

N = 50000
E = 200000
D = 128
CW = 32
NCHUNK = D // CW
BLK = 128
NC = 2
NS = 16
NBLK = 1568
E_PAD = NBLK * BLK
NB2 = NBLK // NS // 2
NBC = NBLK // (NS * NC)
ACC_ROWS = 51200
ROWS_PER_TILE = ACC_ROWS // 16
DCHUNK = 160
NDC = ROWS_PER_TILE // DCHUNK


def _sc_body(whf, whb, whc, sf, df, sb, db, sc2, dc2,
             out_f, out_b, out_c,
             gsrc, gidx, didx, rows, tbuf, zbuf, acc,
             semA0, semA1, semG0, semG1):
    cid = lax.axis_index("c")
    sid = lax.axis_index("s")
    r0 = sid * ROWS_PER_TILE

    zero16 = jnp.zeros((16,), jnp.float32)

    def _init_zbuf(i, _):
        zbuf[i, pl.ds(0, 16)] = zero16
        zbuf[i, pl.ds(16, 16)] = zero16
        return 0

    lax.fori_loop(0, DCHUNK, _init_zbuf, 0)

    def zero_acc():
        for k in range(NDC):
            pltpu.sync_copy(zbuf, acc.at[pl.ds(r0 + k * DCHUNK, DCHUNK)])

    def _mk_gidx(buf, c):
        for j in range(BLK // 16):
            s = gsrc[buf, pl.ds(j * 16, 16)]
            gidx[buf, pl.ds(j * 16, 16)] = s * NCHUNK + c

    def feature_pass(srcp, dstp, whv, oute, c):
        zero_acc()
        plsc.subcore_barrier()

        def body(i, _):
            b0 = (sid + (2 * i) * NS) * BLK
            b1 = (sid + (2 * i + 1) * NS) * BLK
            a00 = pltpu.async_copy(srcp.at[pl.ds(b0, BLK)], gsrc.at[0], semA0)
            a01 = pltpu.async_copy(dstp.at[pl.ds(b0, BLK)], didx.at[0], semA0)
            a10 = pltpu.async_copy(srcp.at[pl.ds(b1, BLK)], gsrc.at[1], semA1)
            a11 = pltpu.async_copy(dstp.at[pl.ds(b1, BLK)], didx.at[1], semA1)
            a00.wait()
            a01.wait()
            _mk_gidx(0, c)
            g0 = pltpu.async_copy(whv.at[gidx.at[0]], rows.at[0], semG0)
            a10.wait()
            a11.wait()
            _mk_gidx(1, c)
            g1 = pltpu.async_copy(whv.at[gidx.at[1]], rows.at[1], semG1)
            g0.wait()
            pltpu.sync_copy(rows.at[0], acc.at[didx.at[0]], add=True)
            g1.wait()
            pltpu.sync_copy(rows.at[1], acc.at[didx.at[1]], add=True)
            return 0

        lax.fori_loop(0, NB2, body, 0)
        plsc.subcore_barrier()
        for k in range(NDC):
            rb = r0 + k * DCHUNK
            pltpu.sync_copy(acc.at[pl.ds(rb, DCHUNK)], tbuf)
            pltpu.sync_copy(tbuf, oute.at[pl.ds(rb, DCHUNK), pl.ds(c * CW, CW)])
        plsc.subcore_barrier()

    for cc in range(NCHUNK // NC):
        c = cc * NC + cid
        feature_pass(sf, df, whf, out_f, c)
        feature_pass(sb, db, whb, out_b, c)
        feature_pass(sc2, dc2, whc, out_c, c)


def _sc_cnt_body(df, db, dc2, cnt, didx, rows, tbuf, zbuf, acc, semA0, semA1):
    cid = lax.axis_index("c")
    sid = lax.axis_index("s")
    r0 = sid * ROWS_PER_TILE

    zero16 = jnp.zeros((16,), jnp.float32)
    one16 = jnp.ones((16,), jnp.float32)

    def _init_zbuf(i, _):
        zbuf[i, pl.ds(0, 16)] = zero16
        zbuf[i, pl.ds(16, 16)] = zero16
        return 0

    lax.fori_loop(0, DCHUNK, _init_zbuf, 0)

    def _init_ones(i, _):
        rows[0, i, pl.ds(0, 16)] = one16
        rows[0, i, pl.ds(16, 16)] = one16
        return 0

    lax.fori_loop(0, BLK, _init_ones, 0)

    def count_pass(dstp, ecol):
        for k in range(NDC):
            pltpu.sync_copy(zbuf, acc.at[pl.ds(r0 + k * DCHUNK, DCHUNK)])
        plsc.subcore_barrier()
        wid = sid * NC + cid

        def body(i, _):
            b0 = (wid + (2 * i) * NS * NC) * BLK
            b1 = (wid + (2 * i + 1) * NS * NC) * BLK
            a0 = pltpu.async_copy(dstp.at[pl.ds(b0, BLK)], didx.at[0], semA0)
            a1 = pltpu.async_copy(dstp.at[pl.ds(b1, BLK)], didx.at[1], semA1)
            a0.wait()
            pltpu.sync_copy(rows.at[0], acc.at[didx.at[0]], add=True)
            a1.wait()
            pltpu.sync_copy(rows.at[0], acc.at[didx.at[1]], add=True)
            return 0

        lax.fori_loop(0, NBC // 2, body, 0)
        bt = (wid + (NBC - 1) * NS * NC) * BLK
        pltpu.sync_copy(dstp.at[pl.ds(bt, BLK)], didx.at[0])
        pltpu.sync_copy(rows.at[0], acc.at[didx.at[0]], add=True)
        plsc.subcore_barrier()
        for k in range(NDC):
            rb = r0 + k * DCHUNK
            pltpu.sync_copy(acc.at[pl.ds(rb, DCHUNK)], tbuf)
            pltpu.sync_copy(tbuf, cnt.at[cid, pl.ds(rb, DCHUNK),
                                         pl.ds(ecol * CW, CW)])
        plsc.subcore_barrier()

    count_pass(df, 0)
    count_pass(db, 1)
    count_pass(dc2, 2)


_sc_cnt = pl.kernel(
    _sc_cnt_body,
    out_type=[jax.ShapeDtypeStruct((NC, ACC_ROWS, D), jnp.float32)],
    mesh=plsc.VectorSubcoreMesh(core_axis_name="c", subcore_axis_name="s",
                                num_cores=NC, num_subcores=NS),
    scratch_types=[
        pltpu.VMEM((2, BLK), jnp.int32),
        pltpu.VMEM((1, BLK, CW), jnp.float32),
        pltpu.VMEM((DCHUNK, CW), jnp.float32),
        pltpu.VMEM((DCHUNK, CW), jnp.float32),
        pltpu.VMEM_SHARED((ACC_ROWS, CW), jnp.float32),
        pltpu.SemaphoreType.DMA,
        pltpu.SemaphoreType.DMA,
    ],
    compiler_params=pltpu.CompilerParams(use_tc_tiling_on_sc=False),
)


_sc_agg = pl.kernel(
    _sc_body,
    out_type=[jax.ShapeDtypeStruct((ACC_ROWS, D), jnp.float32)] * 3,
    mesh=plsc.VectorSubcoreMesh(core_axis_name="c", subcore_axis_name="s",
                                num_cores=NC, num_subcores=NS),
    scratch_types=[
        pltpu.VMEM((2, BLK), jnp.int32),
        pltpu.VMEM((2, BLK), jnp.int32),
        pltpu.VMEM((2, BLK), jnp.int32),
        pltpu.VMEM((2, BLK, CW), jnp.float32),
        pltpu.VMEM((DCHUNK, CW), jnp.float32),
        pltpu.VMEM((DCHUNK, CW), jnp.float32),
        pltpu.VMEM_SHARED((ACC_ROWS, CW), jnp.float32),
        pltpu.SemaphoreType.DMA,
        pltpu.SemaphoreType.DMA,
        pltpu.SemaphoreType.DMA,
        pltpu.SemaphoreType.DMA,
    ],
    compiler_params=pltpu.CompilerParams(use_tc_tiling_on_sc=False),
)


def _lin2_body(x_ref, wf_ref, bf_ref, wb_ref, bb_ref, of_ref, ob_ref):
    x = x_ref[...]
    dn = (((1,), (1,)), ((), ()))
    of_ref[...] = lax.dot_general(x, wf_ref[...], dn,
                                  preferred_element_type=jnp.float32) + bf_ref[...]
    ob_ref[...] = lax.dot_general(x, wb_ref[...], dn,
                                  preferred_element_type=jnp.float32) + bb_ref[...]


def _lin1_body(x_ref, w_ref, b_ref, o_ref):
    x = x_ref[...]
    dn = (((1,), (1,)), ((), ()))
    o_ref[...] = lax.dot_general(x, w_ref[...], dn,
                                 preferred_element_type=jnp.float32) + b_ref[...]


_MB = 5000


def _linear2(x, wf, bf, wb, bb):
    grid = N // _MB
    wspec = pl.BlockSpec((D, D), lambda i: (0, 0))
    bspec = pl.BlockSpec((1, D), lambda i: (0, 0))
    xspec = pl.BlockSpec((_MB, D), lambda i: (i, 0))
    return pl.pallas_call(
        _lin2_body,
        grid=(grid,),
        in_specs=[xspec, wspec, bspec, wspec, bspec],
        out_specs=[xspec, xspec],
        out_shape=[jax.ShapeDtypeStruct((N, D), jnp.float32)] * 2,
    )(x, wf, bf.reshape(1, D), wb, bb.reshape(1, D))


def _linear1(x, w, b):
    grid = N // _MB
    wspec = pl.BlockSpec((D, D), lambda i: (0, 0))
    bspec = pl.BlockSpec((1, D), lambda i: (0, 0))
    xspec = pl.BlockSpec((_MB, D), lambda i: (i, 0))
    return pl.pallas_call(
        _lin1_body,
        grid=(grid,),
        in_specs=[xspec, wspec, bspec],
        out_specs=xspec,
        out_shape=jax.ShapeDtypeStruct((N, D), jnp.float32),
    )(x, w, b.reshape(1, D))


_CB = 5000


def _combine_body(sf_ref, sc_ref, sb_ref, cnt_ref, hu_ref, hi_ref):
    c0 = cnt_ref[0]
    c1 = cnt_ref[1]
    rf = 1.0 / jnp.maximum(c0[:, 0] + c1[:, 0], 1.0)
    rb = 1.0 / jnp.maximum(c0[:, CW] + c1[:, CW], 1.0)
    rc = 1.0 / jnp.maximum(c0[:, 2 * CW] + c1[:, 2 * CW], 1.0)
    hu_ref[...] = (sf_ref[...] * rf[:, None] + sc_ref[...] * rc[:, None]) * 0.5
    hi_ref[...] = sb_ref[...] * rb[:, None]


def _combine(sf, sc2, sb, cnt):
    grid = N // _CB
    sspec = pl.BlockSpec((_CB, D), lambda i: (i, 0))
    cspec = pl.BlockSpec((NC, _CB, D), lambda i: (0, i, 0))
    ospec = pl.BlockSpec((_CB, D), lambda i: (i, 0))
    return pl.pallas_call(
        _combine_body,
        grid=(grid,),
        in_specs=[sspec, sspec, sspec, cspec],
        out_specs=[ospec, ospec],
        out_shape=[jax.ShapeDtypeStruct((N, D), jnp.float32)] * 2,
    )(sf, sc2, sb, cnt)


def _pad_edges(e):
    pad = E_PAD - E
    src = jnp.concatenate([e[0], jnp.zeros((pad,), jnp.int32)])
    dst = jnp.concatenate([e[1], jnp.full((pad,), N, jnp.int32)])
    return src, dst


@jax.jit
def kernel(x_user, x_item, e_follows, e_buys, e_clicks,
           W_follows, b_follows, W_buys, b_buys, W_clicks, b_clicks):
    whf, whb = _linear2(x_user, W_follows, b_follows, W_buys, b_buys)
    whc = _linear1(x_item, W_clicks, b_clicks)
    sf, df = _pad_edges(e_follows)
    sb, db = _pad_edges(e_buys)
    sc2, dc2 = _pad_edges(e_clicks)
    cnt, = _sc_cnt(df, db, dc2)
    out_f, out_b, out_c = _sc_agg(
        whf.reshape(N * NCHUNK, CW), whb.reshape(N * NCHUNK, CW),
        whc.reshape(N * NCHUNK, CW), sf, df, sb, db, sc2, dc2)
    h_user, h_item = _combine(out_f, out_c, out_b, cnt)
    return h_user, h_item


# speedup vs baseline: 1.0025x; 1.0025x over previous
import jax
import jax.numpy as jnp
from jax import lax
from jax.experimental import pallas as pl
from jax.experimental.pallas import tpu as pltpu
from jax.experimental.pallas import tpu_sc as plsc

N = 50000
E = 200000
D = 128
CW = 32
NCHUNK = D // CW
BLK = 128
NC = 2
NS = 16
NBLK = 1568
E_PAD = NBLK * BLK
NB2 = NBLK // NS // 2
NBC = NBLK // (NS * NC)
ACC_ROWS = 51200
ROWS_PER_TILE = ACC_ROWS // 16
DCHUNK = 160
NDC = ROWS_PER_TILE // DCHUNK


def _sc_body(whf, whb, whc, sf, df, sb, db, sc2, dc2,
             out_f, out_b, out_c, cnt,
             gsrc, gidx, didx, rows, tbuf, zbuf, acc,
             semA0, semA1, semG0, semG1):
    cid = lax.axis_index("c")
    sid = lax.axis_index("s")
    r0 = sid * ROWS_PER_TILE

    zero16 = jnp.zeros((16,), jnp.float32)

    def _init_zbuf(i, _):
        zbuf[i, pl.ds(0, 16)] = zero16
        zbuf[i, pl.ds(16, 16)] = zero16
        return 0

    lax.fori_loop(0, DCHUNK, _init_zbuf, 0)

    def zero_acc():
        for k in range(NDC):
            pltpu.sync_copy(zbuf, acc.at[pl.ds(r0 + k * DCHUNK, DCHUNK)])

    def _mk_gidx(buf, c):
        for j in range(BLK // 16):
            s = gsrc[buf, pl.ds(j * 16, 16)]
            gidx[buf, pl.ds(j * 16, 16)] = s * NCHUNK + c

    def feature_pass(srcp, dstp, whv, oute, c):
        zero_acc()
        plsc.subcore_barrier()

        def body(i, _):
            b0 = (sid + (2 * i) * NS) * BLK
            b1 = (sid + (2 * i + 1) * NS) * BLK
            a00 = pltpu.async_copy(srcp.at[pl.ds(b0, BLK)], gsrc.at[0], semA0)
            a01 = pltpu.async_copy(dstp.at[pl.ds(b0, BLK)], didx.at[0], semA0)
            a10 = pltpu.async_copy(srcp.at[pl.ds(b1, BLK)], gsrc.at[1], semA1)
            a11 = pltpu.async_copy(dstp.at[pl.ds(b1, BLK)], didx.at[1], semA1)
            a00.wait()
            a01.wait()
            _mk_gidx(0, c)
            g0 = pltpu.async_copy(whv.at[gidx.at[0]], rows.at[0], semG0)
            a10.wait()
            a11.wait()
            _mk_gidx(1, c)
            g1 = pltpu.async_copy(whv.at[gidx.at[1]], rows.at[1], semG1)
            g0.wait()
            pltpu.sync_copy(rows.at[0], acc.at[didx.at[0]], add=True)
            g1.wait()
            pltpu.sync_copy(rows.at[1], acc.at[didx.at[1]], add=True)
            return 0

        lax.fori_loop(0, NB2, body, 0)
        plsc.subcore_barrier()
        for k in range(NDC):
            rb = r0 + k * DCHUNK
            pltpu.sync_copy(acc.at[pl.ds(rb, DCHUNK)], tbuf)
            pltpu.sync_copy(tbuf, oute.at[pl.ds(rb, DCHUNK), pl.ds(c * CW, CW)])
        plsc.subcore_barrier()

    def count_pass(dstp, ecol):
        zero_acc()
        plsc.subcore_barrier()
        wid = sid * NC + cid

        def body(i, _):
            b0 = (wid + (2 * i) * NS * NC) * BLK
            b1 = (wid + (2 * i + 1) * NS * NC) * BLK
            a0 = pltpu.async_copy(dstp.at[pl.ds(b0, BLK)], didx.at[0], semA0)
            a1 = pltpu.async_copy(dstp.at[pl.ds(b1, BLK)], didx.at[1], semA1)
            a0.wait()
            pltpu.sync_copy(rows.at[0], acc.at[didx.at[0]], add=True)
            a1.wait()
            pltpu.sync_copy(rows.at[0], acc.at[didx.at[1]], add=True)
            return 0

        lax.fori_loop(0, NBC // 2, body, 0)
        bt = (wid + (NBC - 1) * NS * NC) * BLK
        pltpu.sync_copy(dstp.at[pl.ds(bt, BLK)], didx.at[0])
        pltpu.sync_copy(rows.at[0], acc.at[didx.at[0]], add=True)
        plsc.subcore_barrier()
        for k in range(NDC):
            rb = r0 + k * DCHUNK
            pltpu.sync_copy(acc.at[pl.ds(rb, DCHUNK)], tbuf)
            pltpu.sync_copy(tbuf, cnt.at[cid, pl.ds(rb, DCHUNK),
                                         pl.ds(ecol * CW, CW)])
        plsc.subcore_barrier()

    for cc in range(NCHUNK // NC):
        c = cc * NC + cid
        feature_pass(sf, df, whf, out_f, c)
        feature_pass(sb, db, whb, out_b, c)
        feature_pass(sc2, dc2, whc, out_c, c)

    one16 = jnp.ones((16,), jnp.float32)

    def _init_ones(i, _):
        rows[0, i, pl.ds(0, 16)] = one16
        rows[0, i, pl.ds(16, 16)] = one16
        return 0

    lax.fori_loop(0, BLK, _init_ones, 0)
    count_pass(df, 0)
    count_pass(db, 1)
    count_pass(dc2, 2)


_sc_agg = pl.kernel(
    _sc_body,
    out_type=[jax.ShapeDtypeStruct((ACC_ROWS, D), jnp.float32)] * 3
    + [jax.ShapeDtypeStruct((NC, ACC_ROWS, D), jnp.float32)],
    mesh=plsc.VectorSubcoreMesh(core_axis_name="c", subcore_axis_name="s",
                                num_cores=NC, num_subcores=NS),
    scratch_types=[
        pltpu.VMEM((2, BLK), jnp.int32),
        pltpu.VMEM((2, BLK), jnp.int32),
        pltpu.VMEM((2, BLK), jnp.int32),
        pltpu.VMEM((2, BLK, CW), jnp.float32),
        pltpu.VMEM((DCHUNK, CW), jnp.float32),
        pltpu.VMEM((DCHUNK, CW), jnp.float32),
        pltpu.VMEM_SHARED((ACC_ROWS, CW), jnp.float32),
        pltpu.SemaphoreType.DMA,
        pltpu.SemaphoreType.DMA,
        pltpu.SemaphoreType.DMA,
        pltpu.SemaphoreType.DMA,
    ],
    compiler_params=pltpu.CompilerParams(use_tc_tiling_on_sc=False),
)


def _lin2_body(x_ref, wf_ref, bf_ref, wb_ref, bb_ref, of_ref, ob_ref):
    x = x_ref[...]
    dn = (((1,), (1,)), ((), ()))
    of_ref[...] = lax.dot_general(x, wf_ref[...], dn,
                                  preferred_element_type=jnp.float32) + bf_ref[...]
    ob_ref[...] = lax.dot_general(x, wb_ref[...], dn,
                                  preferred_element_type=jnp.float32) + bb_ref[...]


def _lin1_body(x_ref, w_ref, b_ref, o_ref):
    x = x_ref[...]
    dn = (((1,), (1,)), ((), ()))
    o_ref[...] = lax.dot_general(x, w_ref[...], dn,
                                 preferred_element_type=jnp.float32) + b_ref[...]


_MB = 5000


def _linear2(x, wf, bf, wb, bb):
    grid = N // _MB
    wspec = pl.BlockSpec((D, D), lambda i: (0, 0))
    bspec = pl.BlockSpec((1, D), lambda i: (0, 0))
    xspec = pl.BlockSpec((_MB, D), lambda i: (i, 0))
    return pl.pallas_call(
        _lin2_body,
        grid=(grid,),
        in_specs=[xspec, wspec, bspec, wspec, bspec],
        out_specs=[xspec, xspec],
        out_shape=[jax.ShapeDtypeStruct((N, D), jnp.float32)] * 2,
    )(x, wf, bf.reshape(1, D), wb, bb.reshape(1, D))


def _linear1(x, w, b):
    grid = N // _MB
    wspec = pl.BlockSpec((D, D), lambda i: (0, 0))
    bspec = pl.BlockSpec((1, D), lambda i: (0, 0))
    xspec = pl.BlockSpec((_MB, D), lambda i: (i, 0))
    return pl.pallas_call(
        _lin1_body,
        grid=(grid,),
        in_specs=[xspec, wspec, bspec],
        out_specs=xspec,
        out_shape=jax.ShapeDtypeStruct((N, D), jnp.float32),
    )(x, w, b.reshape(1, D))


_CB = 5000


def _combine_body(sf_ref, sc_ref, sb_ref, cnt_ref, hu_ref, hi_ref):
    c0 = cnt_ref[0]
    c1 = cnt_ref[1]
    rf = 1.0 / jnp.maximum(c0[:, 0] + c1[:, 0], 1.0)
    rb = 1.0 / jnp.maximum(c0[:, CW] + c1[:, CW], 1.0)
    rc = 1.0 / jnp.maximum(c0[:, 2 * CW] + c1[:, 2 * CW], 1.0)
    hu_ref[...] = (sf_ref[...] * rf[:, None] + sc_ref[...] * rc[:, None]) * 0.5
    hi_ref[...] = sb_ref[...] * rb[:, None]


def _combine(sf, sc2, sb, cnt):
    grid = N // _CB
    sspec = pl.BlockSpec((_CB, D), lambda i: (i, 0))
    cspec = pl.BlockSpec((NC, _CB, D), lambda i: (0, i, 0))
    ospec = pl.BlockSpec((_CB, D), lambda i: (i, 0))
    return pl.pallas_call(
        _combine_body,
        grid=(grid,),
        in_specs=[sspec, sspec, sspec, cspec],
        out_specs=[ospec, ospec],
        out_shape=[jax.ShapeDtypeStruct((N, D), jnp.float32)] * 2,
    )(sf, sc2, sb, cnt)


def _pad_edges(e):
    pad = E_PAD - E
    src = jnp.concatenate([e[0], jnp.zeros((pad,), jnp.int32)])
    dst = jnp.concatenate([e[1], jnp.full((pad,), N, jnp.int32)])
    return src, dst


@jax.jit
def kernel(x_user, x_item, e_follows, e_buys, e_clicks,
           W_follows, b_follows, W_buys, b_buys, W_clicks, b_clicks):
    whf, whb = _linear2(x_user, W_follows, b_follows, W_buys, b_buys)
    whc = _linear1(x_item, W_clicks, b_clicks)
    sf, df = _pad_edges(e_follows)
    sb, db = _pad_edges(e_buys)
    sc2, dc2 = _pad_edges(e_clicks)
    out_f, out_b, out_c, cnt = _sc_agg(
        whf.reshape(N * NCHUNK, CW), whb.reshape(N * NCHUNK, CW),
        whc.reshape(N * NCHUNK, CW), sf, df, sb, db, sc2, dc2)
    h_user, h_item = _combine(out_f, out_c, out_b, cnt)
    return h_user, h_item
